# Initial kernel scaffold; baseline (speedup 1.0000x reference)
import jax
import jax.numpy as jnp
from jax.experimental import pallas as pl


def kernel(x, pos_encoding):
    raise NotImplementedError("write your pallas kernel here")


# speedup vs baseline: 1.5189x; 1.5189x over previous
import functools

import jax
import jax.numpy as jnp
from jax import lax
from jax.experimental import pallas as pl
from jax.experimental.pallas import tpu as pltpu
from jax.experimental.pallas import tpu_sc as plsc

NUM_EMB = 1000
EMB_DIM = 1024
BATCH = 16384

_info = plsc.get_sparse_core_info()
NC, NS = _info.num_cores, _info.num_subcores
NW = NC * NS
B_PER_W = BATCH // NW
CHUNK = 64
NCH = B_PER_W // CHUNK


def _gather_body(idx_hbm, table_hbm, out_hbm, idx_v, rows_v, sem):
    wid = lax.axis_index("s") * NC + lax.axis_index("c")
    pltpu.sync_copy(idx_hbm.at[wid], idx_v)
    base = wid * B_PER_W
    for ch in range(NCH):
        pltpu.async_copy(table_hbm.at[idx_v.at[ch]], rows_v, sem).wait()
        pltpu.sync_copy(rows_v, out_hbm.at[pl.ds(base + ch * CHUNK, CHUNK)])


_gather = functools.partial(
    pl.kernel,
    mesh=plsc.VectorSubcoreMesh(core_axis_name="c", subcore_axis_name="s"),
    out_type=jax.ShapeDtypeStruct((BATCH, EMB_DIM), jnp.float32),
    scratch_types=[
        pltpu.VMEM((NCH, CHUNK), jnp.int32),
        pltpu.VMEM((CHUNK, EMB_DIM), jnp.float32),
        pltpu.SemaphoreType.DMA,
    ],
)(_gather_body)


@jax.jit
def kernel(x, pos_encoding):
    idx = x.reshape(NW, NCH, CHUNK)
    return _gather(idx, pos_encoding)


# speedup vs baseline: 1.5890x; 1.0462x over previous
import functools

import jax
import jax.numpy as jnp
from jax import lax
from jax.experimental import pallas as pl
from jax.experimental.pallas import tpu as pltpu
from jax.experimental.pallas import tpu_sc as plsc

NUM_EMB = 1000
EMB_DIM = 1024
BATCH = 16384

_info = plsc.get_sparse_core_info()
NC, NS = _info.num_cores, _info.num_subcores
NW = NC * NS
B_PER_W = BATCH // NW
CHUNK = 32
NCH = B_PER_W // CHUNK
NBUF = 3


def _gather_body(idx_hbm, table_hbm, out_hbm, idx_v, rows_v,
                 g0, g1, g2, w0, w1, w2):
    gsems = (g0, g1, g2)
    wsems = (w0, w1, w2)
    wid = lax.axis_index("s") * NC + lax.axis_index("c")
    pltpu.sync_copy(idx_hbm.at[wid], idx_v)
    base = wid * B_PER_W

    gh = [None] * NBUF
    wh = [None] * NBUF
    for ch in range(NBUF):
        b = ch % NBUF
        gh[b] = pltpu.async_copy(table_hbm.at[idx_v.at[ch]], rows_v.at[b],
                                 gsems[b])
    for ch in range(NCH):
        b = ch % NBUF
        gh[b].wait()
        wh[b] = pltpu.async_copy(rows_v.at[b],
                                 out_hbm.at[pl.ds(base + ch * CHUNK, CHUNK)],
                                 wsems[b])
        prev = ch - 1
        if prev >= 0 and prev + NBUF < NCH:
            bp = prev % NBUF
            wh[bp].wait()
            gh[bp] = pltpu.async_copy(table_hbm.at[idx_v.at[prev + NBUF]],
                                      rows_v.at[bp], gsems[bp])
    for ch in range(NCH - NBUF, NCH):
        if ch >= 0:
            wh[ch % NBUF].wait()


_gather = functools.partial(
    pl.kernel,
    mesh=plsc.VectorSubcoreMesh(core_axis_name="c", subcore_axis_name="s"),
    out_type=jax.ShapeDtypeStruct((BATCH, EMB_DIM), jnp.float32),
    scratch_types=[
        pltpu.VMEM((NCH, CHUNK), jnp.int32),
        pltpu.VMEM((NBUF, CHUNK, EMB_DIM), jnp.float32),
        pltpu.SemaphoreType.DMA,
        pltpu.SemaphoreType.DMA,
        pltpu.SemaphoreType.DMA,
        pltpu.SemaphoreType.DMA,
        pltpu.SemaphoreType.DMA,
        pltpu.SemaphoreType.DMA,
    ],
)(_gather_body)


@jax.jit
def kernel(x, pos_encoding):
    idx = x.reshape(NW, NCH, CHUNK)
    return _gather(idx, pos_encoding)
